# Initial kernel scaffold; baseline (speedup 1.0000x reference)
import jax
import jax.numpy as jnp
from jax.experimental import pallas as pl


def kernel(x, x_thild, z, num_atoms, edges, emb, cov, params):
    raise NotImplementedError("write your pallas kernel here")


# speedup vs baseline: 1.4699x; 1.4699x over previous
import functools

import jax
import jax.numpy as jnp
from jax import lax
from jax.experimental import pallas as pl
from jax.experimental.pallas import tpu as pltpu
from jax.experimental.pallas import tpu_sc as plsc

_NC = 2
_NS = 16
_NW = _NC * _NS
_CHUNK = 128
_F32 = jnp.float32


def _sc_mesh():
    return plsc.VectorSubcoreMesh(core_axis_name="c", subcore_axis_name="s")


def _sc_gather_rows(table, idx):
    ep = idx.shape[0]
    width = table.shape[1]
    per_w = ep // _NW
    chunks = per_w // _CHUNK

    @functools.partial(
        pl.kernel,
        mesh=_sc_mesh(),
        compiler_params=pltpu.CompilerParams(use_tc_tiling_on_sc=False),
        out_type=jax.ShapeDtypeStruct((ep, width), _F32),
        scratch_types=[
            pltpu.VMEM((_CHUNK,), jnp.int32),
            pltpu.VMEM((_CHUNK, width), _F32),
            pltpu.SemaphoreType.DMA,
        ],
    )
    def k(table_hbm, idx_hbm, out_hbm, idx_v, rows_v, sem):
        wid = lax.axis_index("s") * _NC + lax.axis_index("c")

        def body(t, carry):
            base = wid * per_w + t * _CHUNK
            pltpu.sync_copy(idx_hbm.at[pl.ds(base, _CHUNK)], idx_v)
            pltpu.async_copy(table_hbm.at[idx_v], rows_v, sem).wait()
            pltpu.sync_copy(rows_v, out_hbm.at[pl.ds(base, _CHUNK)])
            return carry

        lax.fori_loop(0, chunks, body, 0)

    return k(table, idx)


def _sc_scatter_add(c1, c2, sidx, nacc):
    ep = sidx.shape[0]
    per_w = ep // _NW
    chunks = per_w // _CHUNK
    rows_t = nacc // _NS

    @functools.partial(
        pl.kernel,
        mesh=_sc_mesh(),
        compiler_params=pltpu.CompilerParams(use_tc_tiling_on_sc=False),
        out_type=[
            jax.ShapeDtypeStruct((_NC, nacc, 16), _F32),
            jax.ShapeDtypeStruct((_NC, nacc, 32), _F32),
        ],
        scratch_types=[
            pltpu.VMEM((_CHUNK,), jnp.int32),
            pltpu.VMEM((_CHUNK, 16), _F32),
            pltpu.VMEM((_CHUNK, 32), _F32),
            pltpu.VMEM((rows_t, 16), _F32),
            pltpu.VMEM((rows_t, 32), _F32),
            pltpu.VMEM_SHARED((nacc, 16), _F32),
            pltpu.VMEM_SHARED((nacc, 32), _F32),
        ],
    )
    def k(c1_hbm, c2_hbm, sidx_hbm, z1_hbm, z2_hbm, o1_hbm, o2_hbm,
          idx_v, b1v, b2v, w1v, w2v, acc1, acc2):
        cid = lax.axis_index("c")
        sid = lax.axis_index("s")
        wid = sid * _NC + cid

        @pl.when(sid == 0)
        def _init():
            pltpu.sync_copy(z1_hbm, acc1)
            pltpu.sync_copy(z2_hbm, acc2)

        plsc.subcore_barrier()

        def body(t, carry):
            base = wid * per_w + t * _CHUNK
            pltpu.sync_copy(sidx_hbm.at[pl.ds(base, _CHUNK)], idx_v)
            pltpu.sync_copy(c1_hbm.at[pl.ds(base, _CHUNK)], b1v)
            pltpu.sync_copy(c2_hbm.at[pl.ds(base, _CHUNK)], b2v)
            pltpu.sync_copy(b1v, acc1.at[idx_v], add=True)
            pltpu.sync_copy(b2v, acc2.at[idx_v], add=True)
            return carry

        lax.fori_loop(0, chunks, body, 0)

        plsc.subcore_barrier()

        r0 = sid * rows_t
        pltpu.sync_copy(acc1.at[pl.ds(r0, rows_t)], w1v)
        pltpu.sync_copy(w1v, o1_hbm.at[cid, pl.ds(r0, rows_t)])
        pltpu.sync_copy(acc2.at[pl.ds(r0, rows_t)], w2v)
        pltpu.sync_copy(w2v, o2_hbm.at[cid, pl.ds(r0, rows_t)])

    zeros1 = jnp.zeros((nacc, 16), _F32)
    zeros2 = jnp.zeros((nacc, 32), _F32)
    return k(c1, c2, sidx, zeros1, zeros2)


def _tc_prep(h, xt16, w1a_t, w1b_t, b1):
    n = h.shape[0]
    r = 1000
    g = n // r

    def body(h_ref, x_ref, wa_ref, wb_ref, b_ref, t0_ref, t1_ref):
        hb = h_ref[...]
        t0_ref[:, :128] = (
            jnp.dot(hb, wa_ref[...], preferred_element_type=_F32, precision=lax.Precision.HIGHEST) + b_ref[...])
        t0_ref[:, 128:] = -x_ref[...]
        t1_ref[:, :128] = jnp.dot(hb, wb_ref[...], preferred_element_type=_F32, precision=lax.Precision.HIGHEST)
        t1_ref[:, 128:] = x_ref[...]

    return pl.pallas_call(
        body,
        grid=(g,),
        in_specs=[
            pl.BlockSpec((r, 128), lambda i: (i, 0)),
            pl.BlockSpec((r, 16), lambda i: (i, 0)),
            pl.BlockSpec((128, 128), lambda i: (0, 0)),
            pl.BlockSpec((128, 128), lambda i: (0, 0)),
            pl.BlockSpec((1, 128), lambda i: (0, 0)),
        ],
        out_specs=[
            pl.BlockSpec((r, 144), lambda i: (i, 0)),
            pl.BlockSpec((r, 144), lambda i: (i, 0)),
        ],
        out_shape=[jax.ShapeDtypeStruct((n, 144), _F32)] * 2,
    )(h, xt16, w1a_t, w1b_t, b1)


def _tc_edge(g0, g1, w1c, w2_t, b2, ww1_t, bw1, ww2, bw2):
    ep = g0.shape[0]
    r = 2048
    g = ep // r

    def body(g0_ref, g1_ref, w1c_ref, w2_ref, b2_ref, ww1_ref, bw1_ref,
             ww2_ref, bw2_ref, c1_ref, c2_ref):
        s = g0_ref[...] + g1_ref[...]
        pre = s[:, :128]
        t16 = s[:, 128:]
        d = jnp.sqrt(jnp.sum(t16 * t16, axis=1, keepdims=True))
        a1 = jax.nn.silu(pre + d * w1c_ref[...])
        m = jax.nn.silu(
            jnp.dot(a1, w2_ref[...], preferred_element_type=_F32, precision=lax.Precision.HIGHEST) + b2_ref[...])
        t = jax.nn.silu(
            jnp.dot(m, ww1_ref[...], preferred_element_type=_F32, precision=lax.Precision.HIGHEST) + bw1_ref[...])
        w = jnp.sum(t * ww2_ref[...], axis=1, keepdims=True) + bw2_ref[...]
        lane = lax.broadcasted_iota(jnp.int32, (1, 16), 1)
        c1_ref[...] = t16 * w + jnp.where(lane == 3, 1.0, 0.0)
        c2_ref[...] = m

    return pl.pallas_call(
        body,
        grid=(g,),
        in_specs=[
            pl.BlockSpec((r, 144), lambda i: (i, 0)),
            pl.BlockSpec((r, 144), lambda i: (i, 0)),
            pl.BlockSpec((1, 128), lambda i: (0, 0)),
            pl.BlockSpec((128, 32), lambda i: (0, 0)),
            pl.BlockSpec((1, 32), lambda i: (0, 0)),
            pl.BlockSpec((32, 32), lambda i: (0, 0)),
            pl.BlockSpec((1, 32), lambda i: (0, 0)),
            pl.BlockSpec((1, 32), lambda i: (0, 0)),
            pl.BlockSpec((1, 1), lambda i: (0, 0)),
        ],
        out_specs=[
            pl.BlockSpec((r, 16), lambda i: (i, 0)),
            pl.BlockSpec((r, 32), lambda i: (i, 0)),
        ],
        out_shape=[
            jax.ShapeDtypeStruct((ep, 16), _F32),
            jax.ShapeDtypeStruct((ep, 32), _F32),
        ],
    )(g0, g1, w1c, w2_t, b2, ww1_t, bw1, ww2, bw2)


def _tc_node(a1, a2, h, xt16, wn1a_t, wn1b_t, bn1, wn2_t, bn2):
    n = h.shape[0]
    r = 1000
    g = n // r

    def body(a1_ref, a2_ref, h_ref, x_ref, wa_ref, wb_ref, b1_ref, w2_ref,
             b2_ref, h_out, x_out):
        s1 = a1_ref[0] + a1_ref[1]
        s2 = a2_ref[0] + a2_ref[1]
        lane = lax.broadcasted_iota(jnp.int32, (1, 16), 1)
        cnt = jnp.sum(jnp.where(lane == 3, s1, 0.0), axis=1, keepdims=True)
        m_i = s2 / jnp.maximum(cnt, 1.0)
        x_out[...] = x_ref[...] + jnp.where(lane < 3, s1, 0.0)
        hb = h_ref[...]
        t = (jnp.dot(hb, wa_ref[...], preferred_element_type=_F32, precision=lax.Precision.HIGHEST)
             + jnp.dot(m_i, wb_ref[...], preferred_element_type=_F32, precision=lax.Precision.HIGHEST)
             + b1_ref[...])
        h_out[...] = hb + (
            jnp.dot(jax.nn.silu(t), w2_ref[...], preferred_element_type=_F32, precision=lax.Precision.HIGHEST)
            + b2_ref[...])

    return pl.pallas_call(
        body,
        grid=(g,),
        in_specs=[
            pl.BlockSpec((2, r, 16), lambda i: (0, i, 0)),
            pl.BlockSpec((2, r, 32), lambda i: (0, i, 0)),
            pl.BlockSpec((r, 128), lambda i: (i, 0)),
            pl.BlockSpec((r, 16), lambda i: (i, 0)),
            pl.BlockSpec((128, 128), lambda i: (0, 0)),
            pl.BlockSpec((32, 128), lambda i: (0, 0)),
            pl.BlockSpec((1, 128), lambda i: (0, 0)),
            pl.BlockSpec((128, 128), lambda i: (0, 0)),
            pl.BlockSpec((1, 128), lambda i: (0, 0)),
        ],
        out_specs=[
            pl.BlockSpec((r, 128), lambda i: (i, 0)),
            pl.BlockSpec((r, 16), lambda i: (i, 0)),
        ],
        out_shape=[
            jax.ShapeDtypeStruct((n, 128), _F32),
            jax.ShapeDtypeStruct((n, 16), _F32),
        ],
    )(a1, a2, h, xt16, wn1a_t, wn1b_t, bn1, wn2_t, bn2)


def kernel(x, x_thild, z, num_atoms, edges, emb, cov, params):
    del x, num_atoms, cov
    n = x_thild.shape[0]
    e = edges.shape[1]
    nacc = 10240
    step = _NW * _CHUNK
    ep = ((e + step - 1) // step) * step

    e0 = edges[0].astype(jnp.int32)
    e1 = edges[1].astype(jnp.int32)
    pad = ep - e
    eg0 = jnp.concatenate([e0, jnp.zeros((pad,), jnp.int32)])
    eg1 = jnp.concatenate([e1, jnp.zeros((pad,), jnp.int32)])
    es0 = jnp.concatenate([e0, jnp.full((pad,), n, jnp.int32)])

    zp_len = ((n + step - 1) // step) * step
    zp = jnp.concatenate([z.astype(jnp.int32),
                          jnp.zeros((zp_len - n,), jnp.int32)])
    h = _sc_gather_rows(emb, zp)[:n]
    xt16 = jnp.pad(x_thild, ((0, 0), (0, 13)))

    for l in range(2):
        p = params["layer%d" % l]
        w1 = p["edge1"]["W"]
        t0, t1 = _tc_prep(
            h, xt16,
            w1[:, :128].T, w1[:, 128:256].T,
            p["edge1"]["b"].reshape(1, 128))
        g0 = _sc_gather_rows(t0, eg0)
        g1 = _sc_gather_rows(t1, eg1)
        c1, c2 = _tc_edge(
            g0, g1,
            w1[:, 256].reshape(1, 128),
            p["edge2"]["W"].T, p["edge2"]["b"].reshape(1, 32),
            p["w1"]["W"].T, p["w1"]["b"].reshape(1, 32),
            p["w2"]["W"].reshape(1, 32), p["w2"]["b"].reshape(1, 1))
        a1, a2 = _sc_scatter_add(c1, c2, es0, nacc)
        h, xt16 = _tc_node(
            a1[:, :n], a2[:, :n], h, xt16,
            p["node1"]["W"][:, :128].T, p["node1"]["W"][:, 128:].T,
            p["node1"]["b"].reshape(1, 128),
            p["node2"]["W"].T, p["node2"]["b"].reshape(1, 128))

    return xt16[:, :3]
